# Initial kernel scaffold; baseline (speedup 1.0000x reference)
import jax
import jax.numpy as jnp
from jax.experimental import pallas as pl


def kernel(center_table, context_table, center_words, context_words, negative_words):
    raise NotImplementedError("write your pallas kernel here")


# speedup vs baseline: 4.1730x; 4.1730x over previous
import functools

import jax
import jax.numpy as jnp
from jax import lax
from jax.experimental import pallas as pl
from jax.experimental.pallas import tpu as pltpu
from jax.experimental.pallas import tpu_sc as plsc

VOCAB = 1_000_000
D = 64
B = 16384
NNEG = 20

NC = 2
NS = 16
NW = NC * NS
CHUNK = 128

B_PER_W = B // NW
NB_PER_W = B * NNEG // NW


def _sc_gather_body(center_hbm, ctx_hbm, cidx_hbm, xidx_hbm, nidx_hbm,
                    outc_hbm, outx_hbm, outn_hbm, idx_v, rows_v, sem):
    wid = lax.axis_index("s") * NC + lax.axis_index("c")

    def gather_range(table_hbm, idx_hbm, out_hbm, base, nchunks):
        def body(i, carry):
            off = pl.multiple_of(base + i * CHUNK, CHUNK)
            pltpu.sync_copy(idx_hbm.at[pl.ds(off, CHUNK)], idx_v)
            pltpu.async_copy(table_hbm.at[idx_v], rows_v, sem).wait()
            pltpu.sync_copy(rows_v, out_hbm.at[pl.ds(off, CHUNK)])
            return carry
        lax.fori_loop(0, nchunks, body, 0)

    gather_range(center_hbm, cidx_hbm, outc_hbm, wid * B_PER_W,
                 B_PER_W // CHUNK)
    gather_range(ctx_hbm, xidx_hbm, outx_hbm, wid * B_PER_W,
                 B_PER_W // CHUNK)
    gather_range(ctx_hbm, nidx_hbm, outn_hbm, wid * NB_PER_W,
                 NB_PER_W // CHUNK)


@jax.jit
def _sc_gather(center_table, context_table, cidx, xidx, nidx):
    mesh = plsc.VectorSubcoreMesh(core_axis_name="c", subcore_axis_name="s")
    return pl.kernel(
        _sc_gather_body,
        out_type=(
            jax.ShapeDtypeStruct((B, D), jnp.float32),
            jax.ShapeDtypeStruct((B, D), jnp.float32),
            jax.ShapeDtypeStruct((B * NNEG, D), jnp.float32),
        ),
        mesh=mesh,
        scratch_types=[
            pltpu.VMEM((CHUNK,), jnp.int32),
            pltpu.VMEM((CHUNK, D), jnp.float32),
            pltpu.SemaphoreType.DMA,
        ],
        compiler_params=pltpu.CompilerParams(use_tc_tiling_on_sc=False),
    )(center_table, context_table, cidx, xidx, nidx)


ROWS_BLK = 512


def _log_sigmoid(x):
    return jnp.minimum(x, 0.0) - jnp.log(1.0 + jnp.exp(-jnp.abs(x)))


def _tc_loss_body(c_ref, x_ref, n_ref, out_ref):
    pid = pl.program_id(0)
    c = c_ref[...]
    x = x_ref[...]
    n = n_ref[...]
    pos = jnp.sum(c * x, axis=1)
    crep = jnp.reshape(
        jnp.broadcast_to(c[:, None, :], (ROWS_BLK, NNEG, D)),
        (ROWS_BLK * NNEG, D))
    ns = jnp.sum(n * crep, axis=1)
    s = jnp.sum(_log_sigmoid(pos)) + jnp.sum(_log_sigmoid(-ns))

    @pl.when(pid == 0)
    def _():
        out_ref[0, 0] = 0.0

    acc = out_ref[0, 0] + s
    out_ref[0, 0] = acc

    @pl.when(pid == pl.num_programs(0) - 1)
    def _():
        out_ref[0, 0] = -acc / B


@jax.jit
def _tc_loss(center_rows, ctx_rows, neg_rows):
    ng = B // ROWS_BLK
    out = pl.pallas_call(
        _tc_loss_body,
        grid=(ng,),
        in_specs=[
            pl.BlockSpec((ROWS_BLK, D), lambda i: (i, 0)),
            pl.BlockSpec((ROWS_BLK, D), lambda i: (i, 0)),
            pl.BlockSpec((ROWS_BLK * NNEG, D), lambda i: (i, 0)),
        ],
        out_specs=pl.BlockSpec(memory_space=pltpu.SMEM),
        out_shape=jax.ShapeDtypeStruct((1, 1), jnp.float32),
    )(center_rows, ctx_rows, neg_rows)
    return out[0, 0]


def kernel(center_table, context_table, center_words, context_words,
           negative_words):
    cidx = center_words.astype(jnp.int32)
    xidx = context_words.astype(jnp.int32)
    nidx = negative_words.astype(jnp.int32).reshape(-1)
    center_rows, ctx_rows, neg_rows = _sc_gather(
        center_table, context_table, cidx, xidx, nidx)
    return _tc_loss(center_rows, ctx_rows, neg_rows)
